# baseline (speedup 1.0000x reference)
import functools

import jax
import jax.numpy as jnp
from jax import lax
from jax.experimental import pallas as pl
from jax.experimental.pallas import tpu as pltpu
from jax.experimental.pallas import tpu_sc as plsc

N = 10000
E = 320000
IN_D = 128
HID = 64
OUT = 32

NC = 2
NS = 16
NW = NC * NS
CHUNK = 128
E_PAD = 327680
EPT = E_PAD // NW
N_PAD = 10240
ROWS_PT = N_PAD // NS

D1 = HID + 16
D2 = OUT

_BR = 1000
_GRID = N // _BR


def _mm1_body(x_ref, w_ref, b_ref, p_ref, r_ref):
    y = jnp.dot(x_ref[...], w_ref[...], preferred_element_type=jnp.float32)
    ones = jnp.ones((_BR, 16), jnp.float32)
    p_ref[...] = jnp.concatenate([y[:, :HID], ones], axis=1)
    r_ref[...] = y[:, HID:] + b_ref[...]


def _mm1(x, w1cat, b1r):
    return pl.pallas_call(
        _mm1_body,
        grid=(_GRID,),
        in_specs=[
            pl.BlockSpec((_BR, IN_D), lambda i: (i, 0)),
            pl.BlockSpec((IN_D, 2 * HID), lambda i: (0, 0)),
            pl.BlockSpec((1, HID), lambda i: (0, 0)),
        ],
        out_specs=[
            pl.BlockSpec((_BR, D1), lambda i: (i, 0)),
            pl.BlockSpec((_BR, HID), lambda i: (i, 0)),
        ],
        out_shape=[
            jax.ShapeDtypeStruct((N, D1), jnp.float32),
            jax.ShapeDtypeStruct((N, HID), jnp.float32),
        ],
    )(x, w1cat, b1r)


def _mm2_body(acc_ref, r1_ref, w_ref, b_ref, p_ref, r_ref):
    a = acc_ref[0] + acc_ref[1]
    deg = a[:, HID:HID + 1]
    dinv = 1.0 / jnp.maximum(deg, 1.0)
    h = jnp.maximum(a[:, :HID] * dinv + r1_ref[...], 0.0)
    y = jnp.dot(h, w_ref[...], preferred_element_type=jnp.float32)
    p_ref[...] = y[:, :OUT]
    r_ref[...] = jnp.concatenate(
        [y[:, OUT:] + b_ref[...], jnp.broadcast_to(dinv, (_BR, 16))], axis=1)


def _mm2(acc1, r1, w2cat, b2r):
    return pl.pallas_call(
        _mm2_body,
        grid=(_GRID,),
        in_specs=[
            pl.BlockSpec((NC, _BR, D1), lambda i: (0, i, 0)),
            pl.BlockSpec((_BR, HID), lambda i: (i, 0)),
            pl.BlockSpec((HID, 2 * OUT), lambda i: (0, 0)),
            pl.BlockSpec((1, OUT), lambda i: (0, 0)),
        ],
        out_specs=[
            pl.BlockSpec((_BR, D2), lambda i: (i, 0)),
            pl.BlockSpec((_BR, OUT + 16), lambda i: (i, 0)),
        ],
        out_shape=[
            jax.ShapeDtypeStruct((N, D2), jnp.float32),
            jax.ShapeDtypeStruct((N, OUT + 16), jnp.float32),
        ],
    )(acc1, r1, w2cat, b2r)


def _comb_body(acc_ref, r_ref, o_ref):
    a = acc_ref[0] + acc_ref[1]
    dinv = r_ref[:, OUT:OUT + 1]
    o_ref[...] = a * dinv + r_ref[:, :OUT]


def _comb(acc2, r2p):
    return pl.pallas_call(
        _comb_body,
        grid=(_GRID,),
        in_specs=[
            pl.BlockSpec((NC, _BR, D2), lambda i: (0, i, 0)),
            pl.BlockSpec((_BR, OUT + 16), lambda i: (i, 0)),
        ],
        out_specs=pl.BlockSpec((_BR, OUT), lambda i: (i, 0)),
        out_shape=jax.ShapeDtypeStruct((N, OUT), jnp.float32),
    )(acc2, r2p)


NCH = EPT // CHUNK
NBUF = 5
NG = NCH // NBUF


@functools.lru_cache(maxsize=None)
def _make_seg_sum(D):
    mesh = plsc.VectorSubcoreMesh(
        core_axis_name="c", subcore_axis_name="s", num_cores=NC,
        num_subcores=NS)

    @functools.partial(
        pl.kernel,
        out_type=jax.ShapeDtypeStruct((NC, N_PAD, D), jnp.float32),
        mesh=mesh,
        scratch_types=[
            pltpu.VMEM((NCH, CHUNK), jnp.int32),
            pltpu.VMEM((NCH, CHUNK), jnp.int32),
            [pltpu.VMEM((CHUNK, D), jnp.float32) for _ in range(NBUF)],
            pltpu.VMEM_SHARED((N_PAD, D), jnp.float32),
            [pltpu.SemaphoreType.DMA for _ in range(NBUF)],
        ],
        compiler_params=pltpu.CompilerParams(use_tc_tiling_on_sc=False),
    )
    def seg(p_hbm, src_hbm, dst_hbm, z_hbm, out_hbm, idx_s, idx_d, rows,
            acc_sh, gsems):
        c = lax.axis_index("c")
        s = lax.axis_index("s")
        wid = c * NS + s
        rbase = s * ROWS_PT
        pltpu.sync_copy(src_hbm.at[pl.ds(wid * NCH, NCH)], idx_s)
        pltpu.sync_copy(dst_hbm.at[pl.ds(wid * NCH, NCH)], idx_d)
        pltpu.sync_copy(z_hbm.at[pl.ds(rbase, ROWS_PT)],
                        acc_sh.at[pl.ds(rbase, ROWS_PT)])
        plsc.subcore_barrier()

        for b in range(NBUF):
            pltpu.async_copy(p_hbm.at[idx_s.at[b]], rows[b], gsems[b])

        def outer(g, carry):
            for b in range(NBUF):
                i = g * NBUF + b
                pltpu.make_async_copy(p_hbm.at[idx_s.at[i]], rows[b],
                                      gsems[b]).wait()
                pltpu.sync_copy(rows[b], acc_sh.at[idx_d.at[i]], add=True)

                @pl.when(i + NBUF < NCH)
                def _():
                    pltpu.async_copy(p_hbm.at[idx_s.at[i + NBUF]], rows[b],
                                     gsems[b])
            return carry

        lax.fori_loop(0, NG, outer, 0)
        plsc.subcore_barrier()
        pltpu.sync_copy(acc_sh.at[pl.ds(rbase, ROWS_PT)],
                        out_hbm.at[c, pl.ds(rbase, ROWS_PT)])

    return seg


def kernel(x, edge_index, W1_l, W1_r, b1, W2_l, W2_r, b2):
    npad = E_PAD - E
    pad_src = jnp.zeros((npad,), jnp.int32)
    pad_dst = (jnp.arange(npad, dtype=jnp.int32) % (N_PAD - N)) + N
    src = jnp.concatenate([edge_index[0].astype(jnp.int32), pad_src])
    dst = jnp.concatenate([edge_index[1].astype(jnp.int32), pad_dst])
    src = src.reshape(E_PAD // CHUNK, CHUNK)
    dst = dst.reshape(E_PAD // CHUNK, CHUNK)
    w1cat = jnp.concatenate([W1_l.T, W1_r.T], axis=1)
    w2cat = jnp.concatenate([W2_l.T, W2_r.T], axis=1)
    b1r = b1.reshape(1, HID)
    b2r = b2.reshape(1, OUT)
    z1 = jnp.zeros((N_PAD, D1), jnp.float32)
    z2 = jnp.zeros((N_PAD, D2), jnp.float32)

    p1, r1 = _mm1(x, w1cat, b1r)
    acc1 = _make_seg_sum(D1)(p1, src, dst, z1)
    p2, r2p = _mm2(acc1, r1, w2cat, b2r)
    acc2 = _make_seg_sum(D2)(p2, src, dst, z2)
    return _comb(acc2, r2p)


# speedup vs baseline: 2.9212x; 2.9212x over previous
import functools

import jax
import jax.numpy as jnp
from jax import lax
from jax.experimental import pallas as pl
from jax.experimental.pallas import tpu as pltpu
from jax.experimental.pallas import tpu_sc as plsc

N = 10000
E = 320000
IN_D = 128
HID = 64
OUT = 32

NC = 2
NS = 16
NW = NC * NS
CHUNK = 80
EPT = E // NW
N_PAD = 10240
ROWS_PT = N_PAD // NS

D1 = HID + 16
D2 = OUT

_BR = 1000
_GRID = N // _BR


def _mm1_body(x_ref, w_ref, b_ref, p_ref, r_ref):
    y = jnp.dot(x_ref[...], w_ref[...], preferred_element_type=jnp.float32)
    p_ref[...] = y[:, :HID]
    r_ref[...] = y[:, HID:] + b_ref[...]


def _mm1(x, w1cat, b1r):
    return pl.pallas_call(
        _mm1_body,
        grid=(_GRID,),
        in_specs=[
            pl.BlockSpec((_BR, IN_D), lambda i: (i, 0)),
            pl.BlockSpec((IN_D, 2 * HID), lambda i: (0, 0)),
            pl.BlockSpec((1, HID), lambda i: (0, 0)),
        ],
        out_specs=[
            pl.BlockSpec((_BR, HID), lambda i: (i, 0)),
            pl.BlockSpec((_BR, HID), lambda i: (i, 0)),
        ],
        out_shape=[
            jax.ShapeDtypeStruct((N, HID), jnp.float32),
            jax.ShapeDtypeStruct((N, HID), jnp.float32),
        ],
    )(x, w1cat, b1r)


def _mm2_body(acc_ref, deg_ref, r1_ref, w_ref, b_ref, p_ref, r_ref):
    a = acc_ref[0] + acc_ref[1]
    deg = deg_ref[0, :, :1] + deg_ref[1, :, :1]
    dinv = 1.0 / jnp.maximum(deg, 1.0)
    h = jnp.maximum(a * dinv + r1_ref[...], 0.0)
    y = jnp.dot(h, w_ref[...], preferred_element_type=jnp.float32)
    p_ref[...] = y[:, :OUT]
    r_ref[...] = jnp.concatenate(
        [y[:, OUT:] + b_ref[...], jnp.broadcast_to(dinv, (_BR, 16))], axis=1)


def _mm2(acc1, deg1, r1, w2cat, b2r):
    return pl.pallas_call(
        _mm2_body,
        grid=(_GRID,),
        in_specs=[
            pl.BlockSpec((NC, _BR, HID), lambda i: (0, i, 0)),
            pl.BlockSpec((NC, _BR, 16), lambda i: (0, i, 0)),
            pl.BlockSpec((_BR, HID), lambda i: (i, 0)),
            pl.BlockSpec((HID, 2 * OUT), lambda i: (0, 0)),
            pl.BlockSpec((1, OUT), lambda i: (0, 0)),
        ],
        out_specs=[
            pl.BlockSpec((_BR, D2), lambda i: (i, 0)),
            pl.BlockSpec((_BR, OUT + 16), lambda i: (i, 0)),
        ],
        out_shape=[
            jax.ShapeDtypeStruct((N, D2), jnp.float32),
            jax.ShapeDtypeStruct((N, OUT + 16), jnp.float32),
        ],
    )(acc1, deg1, r1, w2cat, b2r)


def _comb_body(acc_ref, r_ref, o_ref):
    a = acc_ref[0] + acc_ref[1]
    dinv = r_ref[:, OUT:OUT + 1]
    o_ref[...] = a * dinv + r_ref[:, :OUT]


def _comb(acc2, r2p):
    return pl.pallas_call(
        _comb_body,
        grid=(_GRID,),
        in_specs=[
            pl.BlockSpec((NC, _BR, D2), lambda i: (0, i, 0)),
            pl.BlockSpec((_BR, OUT + 16), lambda i: (i, 0)),
        ],
        out_specs=pl.BlockSpec((_BR, OUT), lambda i: (i, 0)),
        out_shape=jax.ShapeDtypeStruct((N, OUT), jnp.float32),
    )(acc2, r2p)


NCH = EPT // CHUNK
NBUF = 5
NG = NCH // NBUF


@functools.lru_cache(maxsize=None)
def _make_seg_sum_deg():
    D = HID
    mesh = plsc.VectorSubcoreMesh(
        core_axis_name="c", subcore_axis_name="s", num_cores=NC,
        num_subcores=NS)

    @functools.partial(
        pl.kernel,
        out_type=[
            jax.ShapeDtypeStruct((NC, N_PAD, D), jnp.float32),
            jax.ShapeDtypeStruct((NC, N_PAD, 16), jnp.float32),
        ],
        mesh=mesh,
        scratch_types=[
            pltpu.VMEM((NCH, CHUNK), jnp.int32),
            pltpu.VMEM((NCH, CHUNK), jnp.int32),
            [pltpu.VMEM((CHUNK, D), jnp.float32) for _ in range(NBUF)],
            pltpu.VMEM((CHUNK, 16), jnp.float32),
            pltpu.VMEM_SHARED((N_PAD, D), jnp.float32),
            pltpu.VMEM_SHARED((N_PAD, 16), jnp.float32),
            [pltpu.SemaphoreType.DMA for _ in range(NBUF)],
        ],
        compiler_params=pltpu.CompilerParams(use_tc_tiling_on_sc=False),
    )
    def seg(p_hbm, src_hbm, dst_hbm, z_hbm, zd_hbm, ones_hbm, out_hbm,
            outd_hbm, idx_s, idx_d, rows, ones_v, acc_sh, deg_sh, gsems):
        c = lax.axis_index("c")
        s = lax.axis_index("s")
        wid = c * NS + s
        rbase = s * ROWS_PT
        pltpu.sync_copy(src_hbm.at[pl.ds(wid * NCH, NCH)], idx_s)
        pltpu.sync_copy(dst_hbm.at[pl.ds(wid * NCH, NCH)], idx_d)
        pltpu.sync_copy(ones_hbm, ones_v)
        pltpu.sync_copy(z_hbm.at[pl.ds(rbase, ROWS_PT)],
                        acc_sh.at[pl.ds(rbase, ROWS_PT)])
        pltpu.sync_copy(zd_hbm.at[pl.ds(rbase, ROWS_PT)],
                        deg_sh.at[pl.ds(rbase, ROWS_PT)])
        plsc.subcore_barrier()

        for b in range(NBUF):
            pltpu.async_copy(p_hbm.at[idx_s.at[b]], rows[b], gsems[b])

        def outer(g, carry):
            for b in range(NBUF):
                i = g * NBUF + b
                pltpu.make_async_copy(p_hbm.at[idx_s.at[i]], rows[b],
                                      gsems[b]).wait()
                pltpu.sync_copy(rows[b], acc_sh.at[idx_d.at[i]], add=True)
                pltpu.sync_copy(ones_v, deg_sh.at[idx_d.at[i]], add=True)

                @pl.when(i + NBUF < NCH)
                def _():
                    pltpu.async_copy(p_hbm.at[idx_s.at[i + NBUF]], rows[b],
                                     gsems[b])
            return carry

        lax.fori_loop(0, NG, outer, 0)
        plsc.subcore_barrier()
        pltpu.sync_copy(acc_sh.at[pl.ds(rbase, ROWS_PT)],
                        out_hbm.at[c, pl.ds(rbase, ROWS_PT)])
        pltpu.sync_copy(deg_sh.at[pl.ds(rbase, ROWS_PT)],
                        outd_hbm.at[c, pl.ds(rbase, ROWS_PT)])

    return seg


@functools.lru_cache(maxsize=None)
def _make_seg_sum(D):
    mesh = plsc.VectorSubcoreMesh(
        core_axis_name="c", subcore_axis_name="s", num_cores=NC,
        num_subcores=NS)

    @functools.partial(
        pl.kernel,
        out_type=jax.ShapeDtypeStruct((NC, N_PAD, D), jnp.float32),
        mesh=mesh,
        scratch_types=[
            pltpu.VMEM((NCH, CHUNK), jnp.int32),
            pltpu.VMEM((NCH, CHUNK), jnp.int32),
            [pltpu.VMEM((CHUNK, D), jnp.float32) for _ in range(NBUF)],
            pltpu.VMEM_SHARED((N_PAD, D), jnp.float32),
            [pltpu.SemaphoreType.DMA for _ in range(NBUF)],
        ],
        compiler_params=pltpu.CompilerParams(use_tc_tiling_on_sc=False),
    )
    def seg(p_hbm, src_hbm, dst_hbm, z_hbm, out_hbm, idx_s, idx_d, rows,
            acc_sh, gsems):
        c = lax.axis_index("c")
        s = lax.axis_index("s")
        wid = c * NS + s
        rbase = s * ROWS_PT
        pltpu.sync_copy(src_hbm.at[pl.ds(wid * NCH, NCH)], idx_s)
        pltpu.sync_copy(dst_hbm.at[pl.ds(wid * NCH, NCH)], idx_d)
        pltpu.sync_copy(z_hbm.at[pl.ds(rbase, ROWS_PT)],
                        acc_sh.at[pl.ds(rbase, ROWS_PT)])
        plsc.subcore_barrier()

        for b in range(NBUF):
            pltpu.async_copy(p_hbm.at[idx_s.at[b]], rows[b], gsems[b])

        def outer(g, carry):
            for b in range(NBUF):
                i = g * NBUF + b
                pltpu.make_async_copy(p_hbm.at[idx_s.at[i]], rows[b],
                                      gsems[b]).wait()
                pltpu.sync_copy(rows[b], acc_sh.at[idx_d.at[i]], add=True)

                @pl.when(i + NBUF < NCH)
                def _():
                    pltpu.async_copy(p_hbm.at[idx_s.at[i + NBUF]], rows[b],
                                     gsems[b])
            return carry

        lax.fori_loop(0, NG, outer, 0)
        plsc.subcore_barrier()
        pltpu.sync_copy(acc_sh.at[pl.ds(rbase, ROWS_PT)],
                        out_hbm.at[c, pl.ds(rbase, ROWS_PT)])

    return seg


def kernel(x, edge_index, W1_l, W1_r, b1, W2_l, W2_r, b2):
    src = edge_index[0].astype(jnp.int32).reshape(E // CHUNK, CHUNK)
    dst = edge_index[1].astype(jnp.int32).reshape(E // CHUNK, CHUNK)
    w1cat = jnp.concatenate([W1_l.T, W1_r.T], axis=1)
    w2cat = jnp.concatenate([W2_l.T, W2_r.T], axis=1)
    b1r = b1.reshape(1, HID)
    b2r = b2.reshape(1, OUT)
    z1 = jnp.zeros((N_PAD, HID), jnp.float32)
    zd = jnp.zeros((N_PAD, 16), jnp.float32)
    z2 = jnp.zeros((N_PAD, D2), jnp.float32)
    ones = jnp.ones((CHUNK, 16), jnp.float32)

    p1, r1 = _mm1(x, w1cat, b1r)
    acc1, deg1 = _make_seg_sum_deg()(p1, src, dst, z1, zd, ones)
    p2, r2p = _mm2(acc1, deg1, r1, w2cat, b2r)
    acc2 = _make_seg_sum(D2)(p2, src, dst, z2)
    return _comb(acc2, r2p)


# speedup vs baseline: 2.9541x; 1.0112x over previous
import functools

import jax
import jax.numpy as jnp
from jax import lax
from jax.experimental import pallas as pl
from jax.experimental.pallas import tpu as pltpu
from jax.experimental.pallas import tpu_sc as plsc

N = 10000
E = 320000
IN_D = 128
HID = 64
OUT = 32

NC = 2
NS = 16
NW = NC * NS
CHUNK = 80
EPT = E // NW
N_PAD = 10240
ROWS_PT = N_PAD // NS

D1 = HID + 16
D2 = OUT

_BR = 1000
_GRID = N // _BR


def _mm1_body(x_ref, w_ref, b_ref, p_ref, r_ref):
    y = jnp.dot(x_ref[...], w_ref[...], preferred_element_type=jnp.float32)
    ones = jnp.ones((_BR, 16), jnp.float32)
    p_ref[...] = jnp.concatenate([y[:, :HID], ones], axis=1)
    r_ref[...] = y[:, HID:] + b_ref[...]


def _mm1(x, w1cat, b1r):
    return pl.pallas_call(
        _mm1_body,
        grid=(_GRID,),
        in_specs=[
            pl.BlockSpec((_BR, IN_D), lambda i: (i, 0)),
            pl.BlockSpec((IN_D, 2 * HID), lambda i: (0, 0)),
            pl.BlockSpec((1, HID), lambda i: (0, 0)),
        ],
        out_specs=[
            pl.BlockSpec((_BR, D1), lambda i: (i, 0)),
            pl.BlockSpec((_BR, HID), lambda i: (i, 0)),
        ],
        out_shape=[
            jax.ShapeDtypeStruct((N, D1), jnp.float32),
            jax.ShapeDtypeStruct((N, HID), jnp.float32),
        ],
    )(x, w1cat, b1r)


def _mm2_body(acc_ref, r1_ref, w_ref, b_ref, p_ref, r_ref):
    a = acc_ref[0] + acc_ref[1]
    deg = a[:, HID:HID + 1]
    dinv = 1.0 / jnp.maximum(deg, 1.0)
    h = jnp.maximum(a[:, :HID] * dinv + r1_ref[...], 0.0)
    y = jnp.dot(h, w_ref[...], preferred_element_type=jnp.float32)
    p_ref[...] = y[:, :OUT]
    r_ref[...] = jnp.concatenate(
        [y[:, OUT:] + b_ref[...], jnp.broadcast_to(dinv, (_BR, 16))], axis=1)


def _mm2(acc1, r1, w2cat, b2r):
    return pl.pallas_call(
        _mm2_body,
        grid=(_GRID,),
        in_specs=[
            pl.BlockSpec((NC, _BR, D1), lambda i: (0, i, 0)),
            pl.BlockSpec((_BR, HID), lambda i: (i, 0)),
            pl.BlockSpec((HID, 2 * OUT), lambda i: (0, 0)),
            pl.BlockSpec((1, OUT), lambda i: (0, 0)),
        ],
        out_specs=[
            pl.BlockSpec((_BR, D2), lambda i: (i, 0)),
            pl.BlockSpec((_BR, OUT + 16), lambda i: (i, 0)),
        ],
        out_shape=[
            jax.ShapeDtypeStruct((N, D2), jnp.float32),
            jax.ShapeDtypeStruct((N, OUT + 16), jnp.float32),
        ],
    )(acc1, r1, w2cat, b2r)


def _comb_body(acc_ref, r_ref, o_ref):
    a = acc_ref[0] + acc_ref[1]
    dinv = r_ref[:, OUT:OUT + 1]
    o_ref[...] = a * dinv + r_ref[:, :OUT]


def _comb(acc2, r2p):
    return pl.pallas_call(
        _comb_body,
        grid=(_GRID,),
        in_specs=[
            pl.BlockSpec((NC, _BR, D2), lambda i: (0, i, 0)),
            pl.BlockSpec((_BR, OUT + 16), lambda i: (i, 0)),
        ],
        out_specs=pl.BlockSpec((_BR, OUT), lambda i: (i, 0)),
        out_shape=jax.ShapeDtypeStruct((N, OUT), jnp.float32),
    )(acc2, r2p)


NCH = EPT // CHUNK
NBUF = 5
NG = NCH // NBUF


@functools.lru_cache(maxsize=None)
def _make_seg_sum(D):
    mesh = plsc.VectorSubcoreMesh(
        core_axis_name="c", subcore_axis_name="s", num_cores=NC,
        num_subcores=NS)

    @functools.partial(
        pl.kernel,
        out_type=jax.ShapeDtypeStruct((NC, N_PAD, D), jnp.float32),
        mesh=mesh,
        scratch_types=[
            pltpu.VMEM((NCH, CHUNK), jnp.int32),
            pltpu.VMEM((NCH, CHUNK), jnp.int32),
            [pltpu.VMEM((CHUNK, D), jnp.float32) for _ in range(NBUF)],
            pltpu.VMEM_SHARED((N_PAD, D), jnp.float32),
            [pltpu.SemaphoreType.DMA for _ in range(NBUF)],
        ],
        compiler_params=pltpu.CompilerParams(use_tc_tiling_on_sc=False),
    )
    def seg(p_hbm, src_hbm, dst_hbm, z_hbm, out_hbm, idx_s, idx_d, rows,
            acc_sh, gsems):
        c = lax.axis_index("c")
        s = lax.axis_index("s")
        wid = c * NS + s
        rbase = s * ROWS_PT
        pltpu.sync_copy(src_hbm.at[pl.ds(wid * NCH, NCH)], idx_s)
        pltpu.sync_copy(dst_hbm.at[pl.ds(wid * NCH, NCH)], idx_d)
        pltpu.sync_copy(z_hbm.at[pl.ds(rbase, ROWS_PT)],
                        acc_sh.at[pl.ds(rbase, ROWS_PT)])
        plsc.subcore_barrier()

        for b in range(NBUF):
            pltpu.async_copy(p_hbm.at[idx_s.at[b]], rows[b], gsems[b])

        def outer(g, carry):
            for b in range(NBUF):
                i = g * NBUF + b
                pltpu.make_async_copy(p_hbm.at[idx_s.at[i]], rows[b],
                                      gsems[b]).wait()
                pltpu.sync_copy(rows[b], acc_sh.at[idx_d.at[i]], add=True)

                @pl.when(i + NBUF < NCH)
                def _():
                    pltpu.async_copy(p_hbm.at[idx_s.at[i + NBUF]], rows[b],
                                     gsems[b])
            return carry

        lax.fori_loop(0, NG, outer, 0)
        plsc.subcore_barrier()
        pltpu.sync_copy(acc_sh.at[pl.ds(rbase, ROWS_PT)],
                        out_hbm.at[c, pl.ds(rbase, ROWS_PT)])

    return seg


def kernel(x, edge_index, W1_l, W1_r, b1, W2_l, W2_r, b2):
    src = edge_index[0].astype(jnp.int32).reshape(E // CHUNK, CHUNK)
    dst = edge_index[1].astype(jnp.int32).reshape(E // CHUNK, CHUNK)
    w1cat = jnp.concatenate([W1_l.T, W1_r.T], axis=1)
    w2cat = jnp.concatenate([W2_l.T, W2_r.T], axis=1)
    b1r = b1.reshape(1, HID)
    b2r = b2.reshape(1, OUT)
    z1 = jnp.zeros((N_PAD, D1), jnp.float32)
    z2 = jnp.zeros((N_PAD, D2), jnp.float32)

    p1, r1 = _mm1(x, w1cat, b1r)
    acc1 = _make_seg_sum(D1)(p1, src, dst, z1)
    p2, r2p = _mm2(acc1, r1, w2cat, b2r)
    acc2 = _make_seg_sum(D2)(p2, src, dst, z2)
    return _comb(acc2, r2p)
